# baseline (speedup 1.0000x reference)
import jax
import jax.numpy as jnp
from jax import lax
from jax.experimental import pallas as pl
from jax.experimental.pallas import tpu as pltpu
from jax.experimental.pallas import tpu_sc as plsc

N = 10000
D = 128
E = 320000
NC, NS, L = 2, 16, 16
NW = NC * NS
EPW = E // NW
NP = 10240
NPW = NP // NS


def _sc_body(xf_hbm, ei_hbm, out_hbm,
             sidx_v, didx_v, vals_v, init_i, init_v, acc_v, xcol_s, sem):
    c = lax.axis_index("c")
    s = lax.axis_index("s")
    wid = c * NS + s
    n0 = s * NPW

    lane = lax.iota(jnp.int32, L)

    def mk_idx(i, _):
        node = n0 + i * L + lane
        node = jnp.minimum(node, N - 1)
        init_i[pl.ds(i * L, L)] = node * D
        return 0
    lax.fori_loop(0, NPW // L, mk_idx, 0)
    pltpu.async_copy(xf_hbm.at[init_i], init_v, sem).wait()
    pltpu.sync_copy(init_v, xcol_s.at[pl.ds(n0, NPW)])

    zero = jnp.zeros((L,), jnp.float32)
    def mk_zero(i, _):
        acc_v[pl.ds(i * L, L)] = zero
        return 0
    lax.fori_loop(0, NP // L, mk_zero, 0)

    pltpu.sync_copy(ei_hbm.at[pl.ds(wid * EPW, EPW)], sidx_v)
    pltpu.sync_copy(ei_hbm.at[pl.ds(E + wid * EPW, EPW)], didx_v)

    plsc.subcore_barrier()

    @pl.when(wid == 0)
    def _():
        pltpu.sync_copy(xcol_s, acc_v)

    pltpu.async_copy(xcol_s.at[sidx_v], vals_v, sem).wait()

    def scat(i, _):
        d16 = didx_v[pl.ds(i * L, L)]
        v16 = vals_v[pl.ds(i * L, L)]
        plsc.addupdate_scatter(acc_v, [d16], v16)
        return 0
    lax.fori_loop(0, EPW // L, scat, 0)

    pltpu.sync_copy(acc_v, out_hbm.at[wid])


_sc_kernel = pl.kernel(
    _sc_body,
    out_type=jax.ShapeDtypeStruct((NW, NP), jnp.float32),
    mesh=plsc.VectorSubcoreMesh(core_axis_name="c", subcore_axis_name="s"),
    compiler_params=pltpu.CompilerParams(needs_layout_passes=False),
    scratch_types=[
        pltpu.VMEM((EPW,), jnp.int32),
        pltpu.VMEM((EPW,), jnp.int32),
        pltpu.VMEM((EPW,), jnp.float32),
        pltpu.VMEM((NPW,), jnp.int32),
        pltpu.VMEM((NPW,), jnp.float32),
        pltpu.VMEM((NP,), jnp.float32),
        pltpu.VMEM_SHARED((NP,), jnp.float32),
        pltpu.SemaphoreType.DMA,
    ],
)


def _combine_body(p_ref, o_ref):
    total = jnp.sum(p_ref[...], axis=0)
    o_ref[...] = (total > 0.0).astype(jnp.int32)


_combine = pl.pallas_call(
    _combine_body,
    out_shape=jax.ShapeDtypeStruct((NP // 128, 128), jnp.int32),
)


@jax.jit
def kernel(x, edge_index):
    xf = x.reshape(-1)
    ei = edge_index.astype(jnp.int32).reshape(-1)
    partial = _sc_kernel(xf, ei)
    bits = _combine(partial.reshape(NW, NP // 128, 128))
    return (bits.reshape(-1)[:N]).astype(jnp.int64)


# speedup vs baseline: 1.0412x; 1.0412x over previous
import jax
import jax.numpy as jnp
from jax import lax
from jax.experimental import pallas as pl
from jax.experimental.pallas import tpu as pltpu
from jax.experimental.pallas import tpu_sc as plsc

N = 10000
D = 128
E = 320000
NC, NS, L = 2, 16, 16
NW = NC * NS
EPW = E // NW
NP = 10240
NPW = NP // NS


def _sc_body(xf_hbm, ei_hbm, part_hbm, xcol_hbm, stage_hbm,
             sidx_v, didx_v, init_i, init_v, xcol_v, acc_v, red_v,
             sem, sem2):
    c = lax.axis_index("c")
    s = lax.axis_index("s")
    wid = c * NS + s
    n0 = s * NPW

    lane = lax.iota(jnp.int32, L)

    src_dma = pltpu.async_copy(ei_hbm.at[pl.ds(wid * EPW, EPW)], sidx_v, sem)
    dst_dma = pltpu.async_copy(ei_hbm.at[pl.ds(E + wid * EPW, EPW)], didx_v, sem)

    def mk_idx(i, _):
        node = n0 + i * L + lane
        node = jnp.minimum(node, N - 1)
        init_i[pl.ds(i * L, L)] = node * D
        return 0
    lax.fori_loop(0, NPW // L, mk_idx, 0, unroll=8)
    pltpu.async_copy(xf_hbm.at[init_i], init_v, sem2).wait()
    pltpu.sync_copy(init_v, xcol_hbm.at[c, pl.ds(n0, NPW)])

    zero = jnp.zeros((L,), jnp.float32)
    def mk_zero(i, _):
        acc_v[pl.ds(i * L, L)] = zero
        return 0
    lax.fori_loop(0, NP // L, mk_zero, 0, unroll=8)

    @pl.when(c == 0)
    def _():
        def seed(i, _):
            acc_v[pl.ds(n0 + i * L, L)] = init_v[pl.ds(i * L, L)]
            return 0
        lax.fori_loop(0, NPW // L, seed, 0, unroll=8)

    plsc.subcore_barrier()

    pltpu.sync_copy(xcol_hbm.at[c], xcol_v)
    src_dma.wait()
    dst_dma.wait()

    def edge_step(i, _):
        s16 = sidx_v[pl.ds(i * L, L)]
        v16 = plsc.load_gather(xcol_v, [s16])
        d16 = didx_v[pl.ds(i * L, L)]
        plsc.addupdate_scatter(acc_v, [d16], v16)
        return 0
    lax.fori_loop(0, EPW // L, edge_step, 0, unroll=8)

    pltpu.sync_copy(acc_v, stage_hbm.at[wid])

    plsc.subcore_barrier()

    descs = []
    for t in range(NS):
        descs.append(pltpu.async_copy(
            stage_hbm.at[c * NS + t, pl.ds(n0, NPW)], red_v.at[t], sem))
    for dsc in descs:
        dsc.wait()

    def red_step(q, _):
        acc16 = red_v[0, pl.ds(q * L, L)]
        for t in range(1, NS):
            acc16 = acc16 + red_v[t, pl.ds(q * L, L)]
        init_v[pl.ds(q * L, L)] = acc16
        return 0
    lax.fori_loop(0, NPW // L, red_step, 0, unroll=4)

    pltpu.sync_copy(init_v, part_hbm.at[c, pl.ds(n0, NPW)])


_sc_kernel = pl.kernel(
    _sc_body,
    out_type=(
        jax.ShapeDtypeStruct((NC, NP), jnp.float32),
        jax.ShapeDtypeStruct((NC, NP), jnp.float32),
        jax.ShapeDtypeStruct((NW, NP), jnp.float32),
    ),
    mesh=plsc.VectorSubcoreMesh(core_axis_name="c", subcore_axis_name="s"),
    compiler_params=pltpu.CompilerParams(needs_layout_passes=False),
    scratch_types=[
        pltpu.VMEM((EPW,), jnp.int32),
        pltpu.VMEM((EPW,), jnp.int32),
        pltpu.VMEM((NPW,), jnp.int32),
        pltpu.VMEM((NPW,), jnp.float32),
        pltpu.VMEM((NP,), jnp.float32),
        pltpu.VMEM((NP,), jnp.float32),
        pltpu.VMEM((NS, NPW), jnp.float32),
        pltpu.SemaphoreType.DMA,
        pltpu.SemaphoreType.DMA,
    ],
)


def _combine_body(p_ref, o_ref):
    total = p_ref[0] + p_ref[1]
    o_ref[...] = (total > 0.0).astype(jnp.int32)


_combine = pl.pallas_call(
    _combine_body,
    out_shape=jax.ShapeDtypeStruct((NP // 128, 128), jnp.int32),
)


@jax.jit
def kernel(x, edge_index):
    xf = x.reshape(-1)
    ei = edge_index.astype(jnp.int32).reshape(-1)
    partial, _, _ = _sc_kernel(xf, ei)
    bits = _combine(partial.reshape(NC, NP // 128, 128))
    return (bits.reshape(-1)[:N]).astype(jnp.int64)


# speedup vs baseline: 1.1769x; 1.1303x over previous
import jax
import jax.numpy as jnp
from jax import lax
from jax.experimental import pallas as pl
from jax.experimental.pallas import tpu as pltpu
from jax.experimental.pallas import tpu_sc as plsc

N = 10000
D = 128
E = 320000
NC, NS, L = 2, 16, 16
NW = NC * NS
EPW = E // NW
NP = 10240
NPW = NP // NS


def _sc_body(xf_hbm, ei_hbm, part_hbm, xcol_hbm, stage_hbm,
             sidx_v, didx_v, init_i, init_v, xcol_v, acc_v, red_v,
             sem, sem2):
    c = lax.axis_index("c")
    s = lax.axis_index("s")
    wid = c * NS + s
    n0 = s * NPW

    lane = lax.iota(jnp.int32, L)

    src_dma = pltpu.async_copy(ei_hbm.at[pl.ds(wid * EPW, EPW)], sidx_v, sem)
    dst_dma = pltpu.async_copy(ei_hbm.at[pl.ds(E + wid * EPW, EPW)], didx_v, sem)

    @plsc.parallel_loop(0, NPW // L, unroll=8)
    def mk_idx(i):
        node = n0 + i * L + lane
        node = jnp.minimum(node, N - 1)
        init_i[pl.ds(i * L, L)] = node * D
    pltpu.async_copy(xf_hbm.at[init_i], init_v, sem2).wait()
    pltpu.sync_copy(init_v, xcol_hbm.at[c, pl.ds(n0, NPW)])

    zero = jnp.zeros((L,), jnp.float32)

    @plsc.parallel_loop(0, NP // L, unroll=8)
    def mk_zero(i):
        acc_v[pl.ds(i * L, L)] = zero

    @pl.when(c == 0)
    def _():
        @plsc.parallel_loop(0, NPW // L, unroll=8)
        def seed(i):
            acc_v[pl.ds(n0 + i * L, L)] = init_v[pl.ds(i * L, L)]

    plsc.subcore_barrier()

    pltpu.sync_copy(xcol_hbm.at[c], xcol_v)
    src_dma.wait()
    dst_dma.wait()

    @plsc.parallel_loop(0, EPW // L, unroll=8)
    def edge_step(i):
        s16 = sidx_v[pl.ds(i * L, L)]
        v16 = plsc.load_gather(xcol_v, [s16])
        d16 = didx_v[pl.ds(i * L, L)]
        plsc.addupdate_scatter(acc_v, [d16], v16)

    pltpu.sync_copy(acc_v, stage_hbm.at[wid])

    plsc.subcore_barrier()

    descs = []
    for t in range(NS):
        descs.append(pltpu.async_copy(
            stage_hbm.at[c * NS + t, pl.ds(n0, NPW)], red_v.at[t], sem))
    for dsc in descs:
        dsc.wait()

    @plsc.parallel_loop(0, NPW // L, unroll=4)
    def red_step(q):
        acc16 = red_v[0, pl.ds(q * L, L)]
        for t in range(1, NS):
            acc16 = acc16 + red_v[t, pl.ds(q * L, L)]
        init_v[pl.ds(q * L, L)] = acc16

    pltpu.sync_copy(init_v, part_hbm.at[c, pl.ds(n0, NPW)])


_sc_kernel = pl.kernel(
    _sc_body,
    out_type=(
        jax.ShapeDtypeStruct((NC, NP), jnp.float32),
        jax.ShapeDtypeStruct((NC, NP), jnp.float32),
        jax.ShapeDtypeStruct((NW, NP), jnp.float32),
    ),
    mesh=plsc.VectorSubcoreMesh(core_axis_name="c", subcore_axis_name="s"),
    compiler_params=pltpu.CompilerParams(needs_layout_passes=False),
    scratch_types=[
        pltpu.VMEM((EPW,), jnp.int32),
        pltpu.VMEM((EPW,), jnp.int32),
        pltpu.VMEM((NPW,), jnp.int32),
        pltpu.VMEM((NPW,), jnp.float32),
        pltpu.VMEM((NP,), jnp.float32),
        pltpu.VMEM((NP,), jnp.float32),
        pltpu.VMEM((NS, NPW), jnp.float32),
        pltpu.SemaphoreType.DMA,
        pltpu.SemaphoreType.DMA,
    ],
)


def _combine_body(p_ref, o_ref):
    total = p_ref[0] + p_ref[1]
    o_ref[...] = (total > 0.0).astype(jnp.int32)


_combine = pl.pallas_call(
    _combine_body,
    out_shape=jax.ShapeDtypeStruct((NP // 128, 128), jnp.int32),
)


@jax.jit
def kernel(x, edge_index):
    xf = x.reshape(-1)
    ei = edge_index.astype(jnp.int32).reshape(-1)
    partial, _, _ = _sc_kernel(xf, ei)
    bits = _combine(partial.reshape(NC, NP // 128, 128))
    return (bits.reshape(-1)[:N]).astype(jnp.int64)


# speedup vs baseline: 1.1777x; 1.0007x over previous
import jax
import jax.numpy as jnp
from jax import lax
from jax.experimental import pallas as pl
from jax.experimental.pallas import tpu as pltpu
from jax.experimental.pallas import tpu_sc as plsc

N = 10000
D = 128
E = 320000
NC, NS, L = 2, 16, 16
NW = NC * NS
EPW = E // NW
NP = 10240
NPW = NP // NS


def _sc_body(xf_hbm, ei_hbm, part_hbm, xcol_hbm, stage_hbm,
             sidx_v, didx_v, init_i, init_v, xcol_v, acc_v, red_v,
             sem, sem2):
    c = lax.axis_index("c")
    s = lax.axis_index("s")
    wid = c * NS + s
    n0 = s * NPW

    lane = lax.iota(jnp.int32, L)

    src_dma = pltpu.async_copy(ei_hbm.at[pl.ds(wid * EPW, EPW)], sidx_v, sem)
    dst_dma = pltpu.async_copy(ei_hbm.at[pl.ds(E + wid * EPW, EPW)], didx_v, sem)

    with jax.named_scope("ph1_init"):
        @plsc.parallel_loop(0, NPW // L, unroll=8)
        def mk_idx(i):
            node = n0 + i * L + lane
            node = jnp.minimum(node, N - 1)
            init_i[pl.ds(i * L, L)] = node * D
        pltpu.async_copy(xf_hbm.at[init_i], init_v, sem2).wait()
        pltpu.sync_copy(init_v, xcol_hbm.at[c, pl.ds(n0, NPW)])

    with jax.named_scope("ph2_zero"):
        zero = jnp.zeros((L,), jnp.float32)

        @plsc.parallel_loop(0, NP // L, unroll=8)
        def mk_zero(i):
            acc_v[pl.ds(i * L, L)] = zero

        @pl.when(c == 0)
        def _():
            @plsc.parallel_loop(0, NPW // L, unroll=8)
            def seed(i):
                acc_v[pl.ds(n0 + i * L, L)] = init_v[pl.ds(i * L, L)]

    with jax.named_scope("ph3_barrier"):
        plsc.subcore_barrier()

    with jax.named_scope("ph4_bcast"):
        pltpu.sync_copy(xcol_hbm.at[c], xcol_v)
        src_dma.wait()
        dst_dma.wait()

    with jax.named_scope("ph5_edges"):
        @plsc.parallel_loop(0, EPW // L, unroll=8)
        def edge_step(i):
            s16 = sidx_v[pl.ds(i * L, L)]
            v16 = plsc.load_gather(xcol_v, [s16])
            d16 = didx_v[pl.ds(i * L, L)]
            plsc.addupdate_scatter(acc_v, [d16], v16)

    with jax.named_scope("ph6_stage"):
        pltpu.sync_copy(acc_v, stage_hbm.at[wid])

    with jax.named_scope("ph7_barrier"):
        plsc.subcore_barrier()

    with jax.named_scope("ph8_reduce"):
        descs = []
        for t in range(NS):
            descs.append(pltpu.async_copy(
                stage_hbm.at[c * NS + t, pl.ds(n0, NPW)], red_v.at[t], sem))
        for dsc in descs:
            dsc.wait()

        @plsc.parallel_loop(0, NPW // L, unroll=4)
        def red_step(q):
            acc16 = red_v[0, pl.ds(q * L, L)]
            for t in range(1, NS):
                acc16 = acc16 + red_v[t, pl.ds(q * L, L)]
            init_v[pl.ds(q * L, L)] = acc16

        pltpu.sync_copy(init_v, part_hbm.at[c, pl.ds(n0, NPW)])


_sc_kernel = pl.kernel(
    _sc_body,
    out_type=(
        jax.ShapeDtypeStruct((NC, NP), jnp.float32),
        jax.ShapeDtypeStruct((NC, NP), jnp.float32),
        jax.ShapeDtypeStruct((NW, NP), jnp.float32),
    ),
    mesh=plsc.VectorSubcoreMesh(core_axis_name="c", subcore_axis_name="s"),
    compiler_params=pltpu.CompilerParams(needs_layout_passes=False),
    scratch_types=[
        pltpu.VMEM((EPW,), jnp.int32),
        pltpu.VMEM((EPW,), jnp.int32),
        pltpu.VMEM((NPW,), jnp.int32),
        pltpu.VMEM((NPW,), jnp.float32),
        pltpu.VMEM((NP,), jnp.float32),
        pltpu.VMEM((NP,), jnp.float32),
        pltpu.VMEM((NS, NPW), jnp.float32),
        pltpu.SemaphoreType.DMA,
        pltpu.SemaphoreType.DMA,
    ],
)


def _combine_body(p_ref, o_ref):
    total = p_ref[0] + p_ref[1]
    o_ref[...] = (total > 0.0).astype(jnp.int32)


_combine = pl.pallas_call(
    _combine_body,
    out_shape=jax.ShapeDtypeStruct((NP // 128, 128), jnp.int32),
)


@jax.jit
def kernel(x, edge_index):
    xf = x.reshape(-1)
    ei = edge_index.astype(jnp.int32).reshape(-1)
    partial, _, _ = _sc_kernel(xf, ei)
    bits = _combine(partial.reshape(NC, NP // 128, 128))
    return (bits.reshape(-1)[:N]).astype(jnp.int64)


# speedup vs baseline: 1.2255x; 1.0406x over previous
import jax
import jax.numpy as jnp
from jax import lax
from jax.experimental import pallas as pl
from jax.experimental.pallas import tpu as pltpu
from jax.experimental.pallas import tpu_sc as plsc

N = 10000
D = 128
E = 320000
NC, NS, L = 2, 16, 16
NW = NC * NS
EPW = E // NW
NP = 10240
NPW = NP // NS


def _sc_body(xf_hbm, ei_hbm, part_hbm, xcol_hbm, stage_hbm,
             sidx_v, didx_v, init_i, init_v, xcol_v, acc_v, red_v,
             sem, sem2):
    c = lax.axis_index("c")
    s = lax.axis_index("s")
    wid = c * NS + s
    n0 = s * NPW

    lane = lax.iota(jnp.int32, L)

    src_dma = pltpu.async_copy(ei_hbm.at[pl.ds(wid * EPW, EPW)], sidx_v, sem)
    dst_dma = pltpu.async_copy(ei_hbm.at[pl.ds(E + wid * EPW, EPW)], didx_v, sem)

    with jax.named_scope("ph1_init"):
        @plsc.parallel_loop(0, NPW // L, unroll=8)
        def mk_idx(i):
            node = n0 + i * L + lane
            node = jnp.where(node >= N, node - N, node)
            init_i[pl.ds(i * L, L)] = node * D
        pltpu.async_copy(xf_hbm.at[init_i], init_v, sem2).wait()
        pltpu.sync_copy(init_v, xcol_hbm.at[c, pl.ds(n0, NPW)])

    with jax.named_scope("ph2_zero"):
        zero = jnp.zeros((L,), jnp.float32)

        @plsc.parallel_loop(0, NP // L, unroll=8)
        def mk_zero(i):
            acc_v[pl.ds(i * L, L)] = zero

        @pl.when(c == 0)
        def _():
            @plsc.parallel_loop(0, NPW // L, unroll=8)
            def seed(i):
                acc_v[pl.ds(n0 + i * L, L)] = init_v[pl.ds(i * L, L)]

    with jax.named_scope("ph3_barrier"):
        plsc.subcore_barrier()

    with jax.named_scope("ph4_bcast"):
        bdescs = []
        for t in range(NS):
            chunk = lax.rem(s + t, NS)
            off = chunk * NPW
            bdescs.append(pltpu.async_copy(
                xcol_hbm.at[c, pl.ds(off, NPW)],
                xcol_v.at[pl.ds(off, NPW)], sem2))
        for dsc in bdescs:
            dsc.wait()
        src_dma.wait()
        dst_dma.wait()

    with jax.named_scope("ph5_edges"):
        @plsc.parallel_loop(0, EPW // L, unroll=8)
        def edge_step(i):
            s16 = sidx_v[pl.ds(i * L, L)]
            v16 = plsc.load_gather(xcol_v, [s16])
            d16 = didx_v[pl.ds(i * L, L)]
            plsc.addupdate_scatter(acc_v, [d16], v16)

    with jax.named_scope("ph6_stage"):
        pltpu.sync_copy(acc_v, stage_hbm.at[wid])

    with jax.named_scope("ph7_barrier"):
        plsc.subcore_barrier()

    with jax.named_scope("ph8_reduce"):
        descs = []
        for t in range(NS):
            descs.append(pltpu.async_copy(
                stage_hbm.at[c * NS + t, pl.ds(n0, NPW)], red_v.at[t], sem))
        for dsc in descs:
            dsc.wait()

        @plsc.parallel_loop(0, NPW // L, unroll=4)
        def red_step(q):
            acc16 = red_v[0, pl.ds(q * L, L)]
            for t in range(1, NS):
                acc16 = acc16 + red_v[t, pl.ds(q * L, L)]
            init_v[pl.ds(q * L, L)] = acc16

        pltpu.sync_copy(init_v, part_hbm.at[c, pl.ds(n0, NPW)])


_sc_kernel = pl.kernel(
    _sc_body,
    out_type=(
        jax.ShapeDtypeStruct((NC, NP), jnp.float32),
        jax.ShapeDtypeStruct((NC, NP), jnp.float32),
        jax.ShapeDtypeStruct((NW, NP), jnp.float32),
    ),
    mesh=plsc.VectorSubcoreMesh(core_axis_name="c", subcore_axis_name="s"),
    compiler_params=pltpu.CompilerParams(needs_layout_passes=False),
    scratch_types=[
        pltpu.VMEM((EPW,), jnp.int32),
        pltpu.VMEM((EPW,), jnp.int32),
        pltpu.VMEM((NPW,), jnp.int32),
        pltpu.VMEM((NPW,), jnp.float32),
        pltpu.VMEM((NP,), jnp.float32),
        pltpu.VMEM((NP,), jnp.float32),
        pltpu.VMEM((NS, NPW), jnp.float32),
        pltpu.SemaphoreType.DMA,
        pltpu.SemaphoreType.DMA,
    ],
)


def _combine_body(p_ref, o_ref):
    total = p_ref[pl.ds(0, NP)] + p_ref[pl.ds(NP, NP)]
    o_ref[...] = (total > 0.0).astype(jnp.int32)


_combine = pl.pallas_call(
    _combine_body,
    out_shape=jax.ShapeDtypeStruct((NP,), jnp.int32),
)


@jax.jit
def kernel(x, edge_index):
    xf = x.reshape(-1)
    ei = edge_index.astype(jnp.int32).reshape(-1)
    partial, _, _ = _sc_kernel(xf, ei)
    bits = _combine(partial.reshape(-1))
    return bits[:N].astype(jnp.int64)


# speedup vs baseline: 1.2659x; 1.0330x over previous
import jax
import jax.numpy as jnp
from jax import lax
from jax.experimental import pallas as pl
from jax.experimental.pallas import tpu as pltpu
from jax.experimental.pallas import tpu_sc as plsc

N = 10000
D = 128
E = 320000
NC, NS, L = 2, 16, 16
NW = NC * NS
EPW = 10112
EPL = E - (NW - 1) * EPW
NP = 10240
NPW = NP // NS


def _sc_body(xf_hbm, ei_hbm, part_hbm, xcol_hbm, stage_hbm,
             both_v, init_i, init_v, xcol_v, acc_v, red_v,
             sem, sem2):
    c = lax.axis_index("c")
    s = lax.axis_index("s")
    wid = c * NS + s
    n0 = s * NPW

    lane = lax.iota(jnp.int32, L)

    with jax.named_scope("ph0_edges_dma"):
        @pl.when(wid < NW - 1)
        def _():
            pltpu.sync_copy(ei_hbm.at[:, pl.ds(wid * EPW, EPW)], both_v)

        @pl.when(wid == NW - 1)
        def _():
            pltpu.sync_copy(ei_hbm.at[:, pl.ds((NW - 1) * EPW, EPL)],
                            both_v.at[:, pl.ds(0, EPL)])

    with jax.named_scope("ph1_init"):
        @plsc.parallel_loop(0, NPW // L, unroll=8)
        def mk_idx(i):
            node = n0 + i * L + lane
            node = jnp.where(node >= N, node - N, node)
            init_i[pl.ds(i * L, L)] = node * D
        pltpu.async_copy(xf_hbm.at[init_i], init_v, sem2).wait()
        pltpu.sync_copy(init_v, xcol_hbm.at[c, pl.ds(n0, NPW)])

    with jax.named_scope("ph2_zero"):
        zero = jnp.zeros((L,), jnp.float32)

        @plsc.parallel_loop(0, NP // L, unroll=8)
        def mk_zero(i):
            acc_v[pl.ds(i * L, L)] = zero

        @pl.when(c == 0)
        def _():
            @plsc.parallel_loop(0, NPW // L, unroll=8)
            def seed(i):
                acc_v[pl.ds(n0 + i * L, L)] = init_v[pl.ds(i * L, L)]

    with jax.named_scope("ph3_barrier"):
        plsc.subcore_barrier()

    with jax.named_scope("ph4_bcast"):
        bdescs = []
        for t in range(NS):
            chunk = lax.rem(s + t, NS)
            off = chunk * NPW
            bdescs.append(pltpu.async_copy(
                xcol_hbm.at[c, pl.ds(off, NPW)],
                xcol_v.at[pl.ds(off, NPW)], sem2))
        for dsc in bdescs:
            dsc.wait()

    n_edges = jnp.where(wid < NW - 1, EPW, EPL)
    with jax.named_scope("ph5_edges"):
        @plsc.parallel_loop(0, EPW // L, unroll=8)
        def edge_step(i):
            m16 = (i * L + lane) < n_edges
            s16 = both_v[0, pl.ds(i * L, L)]
            v16 = plsc.load_gather(xcol_v, [s16], mask=m16)
            d16 = both_v[1, pl.ds(i * L, L)]
            plsc.addupdate_scatter(acc_v, [d16], v16, mask=m16)

    with jax.named_scope("ph6_stage"):
        pltpu.sync_copy(acc_v, stage_hbm.at[wid])

    with jax.named_scope("ph7_barrier"):
        plsc.subcore_barrier()

    with jax.named_scope("ph8_reduce"):
        descs = []
        for t in range(NS):
            descs.append(pltpu.async_copy(
                stage_hbm.at[c * NS + t, pl.ds(n0, NPW)], red_v.at[t], sem))
        for dsc in descs:
            dsc.wait()

        @plsc.parallel_loop(0, NPW // L, unroll=4)
        def red_step(q):
            acc16 = red_v[0, pl.ds(q * L, L)]
            for t in range(1, NS):
                acc16 = acc16 + red_v[t, pl.ds(q * L, L)]
            init_v[pl.ds(q * L, L)] = acc16

        pltpu.sync_copy(init_v, part_hbm.at[c, pl.ds(n0, NPW)])


_sc_kernel = pl.kernel(
    _sc_body,
    out_type=(
        jax.ShapeDtypeStruct((NC, NP), jnp.float32),
        jax.ShapeDtypeStruct((NC, NP), jnp.float32),
        jax.ShapeDtypeStruct((NW, NP), jnp.float32),
    ),
    mesh=plsc.VectorSubcoreMesh(core_axis_name="c", subcore_axis_name="s"),
    compiler_params=pltpu.CompilerParams(needs_layout_passes=False),
    scratch_types=[
        pltpu.VMEM((2, EPW), jnp.int32),
        pltpu.VMEM((NPW,), jnp.int32),
        pltpu.VMEM((NPW,), jnp.float32),
        pltpu.VMEM((NP,), jnp.float32),
        pltpu.VMEM((NP,), jnp.float32),
        pltpu.VMEM((NS, NPW), jnp.float32),
        pltpu.SemaphoreType.DMA,
        pltpu.SemaphoreType.DMA,
    ],
)


def _combine_body(p_ref, o_ref):
    total = p_ref[pl.ds(0, NP)] + p_ref[pl.ds(NP, NP)]
    o_ref[...] = (total > 0.0).astype(jnp.int32)


_combine = pl.pallas_call(
    _combine_body,
    out_shape=jax.ShapeDtypeStruct((NP,), jnp.int32),
)


@jax.jit
def kernel(x, edge_index):
    xf = x.reshape(-1)
    ei = edge_index.astype(jnp.int32)
    partial, _, _ = _sc_kernel(xf, ei)
    bits = _combine(partial.reshape(-1))
    return bits[:N].astype(jnp.int64)
